# baseline (speedup 1.0000x reference)
import functools

import jax
import jax.numpy as jnp
from jax import lax
from jax.experimental import pallas as pl
from jax.experimental.pallas import tpu as pltpu
from jax.experimental.pallas import tpu_sc as plsc

VOCAB = 1000000
DIM = 64
B, L = 4096, 200
NTOK = B * L

NC, NS = 2, 16
NW = NC * NS
CB = 128
PER_W = NTOK // NW
NCHUNK = PER_W // CB
NBUF = 5
LOOK = 2
NGROUP = NCHUNK // NBUF

TBLK = (VOCAB + 127) // 128
VPAD = TBLK * 128
NSTEP = (TBLK + NW - 1) // NW


def _sc_detile(Wt):
    mesh = plsc.VectorSubcoreMesh(core_axis_name="c", subcore_axis_name="s")

    @functools.partial(
        pl.kernel,
        mesh=mesh,
        compiler_params=pltpu.CompilerParams(
            use_tc_tiling_on_sc=True, needs_layout_passes=False),
        out_type=jax.ShapeDtypeStruct((VPAD, 128), jnp.float32),
        scratch_types=[
            pltpu.VMEM((2, DIM, 129), jnp.float32),
            pltpu.VMEM((2, 128, 128), jnp.float32),
        ] + [pltpu.SemaphoreType.DMA] * 4,
    )
    def k(wt_hbm, wp_hbm, blk_v, stg_v, sg0, sg1, sp0, sp1):
        sems_g = (sg0, sg1)
        sems_p = (sp0, sp1)
        wid = lax.axis_index("s") * NC + lax.axis_index("c")
        iota = lax.iota(jnp.int32, 16)
        rows_g = [iota + 16 * g for g in range(4)]

        def start_get(i, s):
            c = i * NW + wid

            @pl.when(c < TBLK)
            def _():
                pltpu.async_copy(wt_hbm.at[:, pl.ds(c * 128, 128)],
                                 blk_v.at[s, :, pl.ds(0, 128)], sems_g[s])

        def wait_get(i, s):
            c = i * NW + wid

            @pl.when(c < TBLK)
            def _():
                pltpu.make_async_copy(wt_hbm.at[:, pl.ds(0, 128)],
                                      blk_v.at[s, :, pl.ds(0, 128)],
                                      sems_g[s]).wait()

        def transpose(i, s):
            c = i * NW + wid

            @pl.when(c < TBLK)
            def _():
                blk = blk_v.at[s]
                stg = stg_v.at[s]

                def body(q, carry):
                    for dv in range(4):
                        v = q * 4 + dv
                        vvec = jnp.full((16,), v, jnp.int32)
                        for g in range(4):
                            vec = plsc.load_gather(blk, [rows_g[g], vvec])
                            stg[v, pl.ds(g * 16, 16)] = vec
                    return carry

                lax.fori_loop(0, 32, body, 0)

        def start_put(i, s):
            c = i * NW + wid

            @pl.when(c < TBLK)
            def _():
                pltpu.async_copy(stg_v.at[s],
                                 wp_hbm.at[pl.ds(c * 128, 128)], sems_p[s])

        def wait_put(i, s):
            c = i * NW + wid

            @pl.when(c < TBLK)
            def _():
                pltpu.make_async_copy(stg_v.at[s],
                                      wp_hbm.at[pl.ds(0, 128)],
                                      sems_p[s]).wait()

        def step(i, s, first_pair):
            wait_get(i, s)
            start_get(i + 1, 1 - s)
            if not first_pair:
                wait_put(i - 2, s)
            transpose(i, s)
            start_put(i, s)

        start_get(0, 0)
        step(0, 0, True)
        step(1, 1, True)
        wait_put(0, 0)
        wait_get(2, 0)
        start_get(3, 1)
        transpose(2, 0)
        start_put(2, 0)

        def pair(g, carry):
            i = 2 * g + 1
            step(i, 1, False)
            step(i + 1, 0, False)
            return carry

        lax.fori_loop(1, 121, pair, 0)
        step(243, 1, False)
        step(244, 0, False)
        wait_put(243, 1)
        wait_put(244, 0)

    return k(Wt)


def _sc_gather(idx3, table):
    mesh = plsc.VectorSubcoreMesh(core_axis_name="c", subcore_axis_name="s")

    @functools.partial(
        pl.kernel,
        mesh=mesh,
        compiler_params=pltpu.CompilerParams(use_tc_tiling_on_sc=False),
        out_type=jax.ShapeDtypeStruct((NTOK, 128), jnp.float32),
        scratch_types=[
            pltpu.VMEM((NCHUNK, CB), jnp.int32),
            pltpu.VMEM((NBUF, CB, 128), jnp.float32),
        ] + [pltpu.SemaphoreType.DMA] * (2 * NBUF),
    )
    def k(idx_hbm, w_hbm, out_hbm, idx_v, rows_v, *sems):
        sem_g, sem_p = sems[:NBUF], sems[NBUF:]
        wid = lax.axis_index("s") * NC + lax.axis_index("c")
        base = wid * PER_W
        pltpu.sync_copy(idx_hbm.at[wid], idx_v)

        def gather(j, b):
            pltpu.async_copy(w_hbm.at[idx_v.at[j]], rows_v.at[b], sem_g[b])

        def wait_gather(j, b):
            pltpu.make_async_copy(
                w_hbm.at[idx_v.at[j]], rows_v.at[b], sem_g[b]).wait()

        def put(j, b):
            pltpu.async_copy(
                rows_v.at[b, :, pl.ds(0, DIM)],
                out_hbm.at[pl.ds(base + j * CB, CB), pl.ds(0, DIM)], sem_p[b])

        def wait_put(b):
            pltpu.make_async_copy(
                rows_v.at[b, :, pl.ds(0, DIM)],
                out_hbm.at[pl.ds(base, CB), pl.ds(0, DIM)], sem_p[b]).wait()

        for t in range(LOOK):
            gather(t, t)
        for b in range(NBUF):
            wait_gather(b, b)
            put(b, b)
            jn = b + LOOK
            bn = jn % NBUF
            if jn >= NBUF:
                wait_put(bn)
            gather(jn, bn)

        def group(g, c):
            j0 = g * NBUF
            for b in range(NBUF):
                j = j0 + b
                wait_gather(j, b)
                put(j, b)
                bn = (b + LOOK) % NBUF
                wait_put(bn)
                gather(j + LOOK, bn)
            return c

        lax.fori_loop(1, NGROUP - 1, group, 0)

        j0 = NCHUNK - NBUF
        for b in range(NBUF):
            j = j0 + b
            wait_gather(j, b)
            put(j, b)
            jn = j + LOOK
            if jn < NCHUNK:
                bn = (b + LOOK) % NBUF
                wait_put(bn)
                gather(jn, bn)
        for b in range(NBUF):
            wait_put(b)

    return k(idx3, table)


def kernel(input_word, W):
    idx3 = input_word.reshape(NW, NCHUNK, CB)
    Wp = _sc_detile(W.T)
    out = _sc_gather(idx3, Wp)
    return out[:, :DIM].reshape(B, L, DIM)


# speedup vs baseline: 1.5793x; 1.5793x over previous
import functools

import jax
import jax.numpy as jnp
from jax import lax
from jax.experimental import pallas as pl
from jax.experimental.pallas import tpu as pltpu
from jax.experimental.pallas import tpu_sc as plsc

VOCAB = 1000000
DIM = 64
B, L = 4096, 200
NTOK = B * L

NC, NS = 2, 16
NW = NC * NS
CB = 128
PER_W = NTOK // NW
NCHUNK = PER_W // CB
NBUF = 5
LOOK = 2
NGROUP = NCHUNK // NBUF

TBLK = (VOCAB + 127) // 128
VPAD = TBLK * 128
NSTEP = (TBLK + NW - 1) // NW


def _sc_detile(Wt):
    mesh = plsc.VectorSubcoreMesh(core_axis_name="c", subcore_axis_name="s")

    @functools.partial(
        pl.kernel,
        mesh=mesh,
        compiler_params=pltpu.CompilerParams(
            use_tc_tiling_on_sc=True, needs_layout_passes=False),
        out_type=jax.ShapeDtypeStruct((VPAD, 128), jnp.float32),
        scratch_types=[
            pltpu.VMEM((2, DIM, 129), jnp.float32),
            pltpu.VMEM((2, 128, 128), jnp.float32),
        ] + [pltpu.SemaphoreType.DMA] * 4,
    )
    def k(wt_hbm, wp_hbm, blk_v, stg_v, sg0, sg1, sp0, sp1):
        sems_g = (sg0, sg1)
        sems_p = (sp0, sp1)
        wid = lax.axis_index("s") * NC + lax.axis_index("c")
        iota = lax.iota(jnp.int32, 16)
        rows_g = [iota + 16 * g for g in range(4)]

        def start_get(i, s):
            c = i * NW + wid

            @pl.when(c < TBLK)
            def _():
                pltpu.async_copy(wt_hbm.at[:, pl.ds(c * 128, 128)],
                                 blk_v.at[s, :, pl.ds(0, 128)], sems_g[s])

        def wait_get(i, s):
            c = i * NW + wid

            @pl.when(c < TBLK)
            def _():
                pltpu.make_async_copy(wt_hbm.at[:, pl.ds(0, 128)],
                                      blk_v.at[s, :, pl.ds(0, 128)],
                                      sems_g[s]).wait()

        def transpose(i, s):
            c = i * NW + wid

            @pl.when(c < TBLK)
            def _():
                blk = blk_v.at[s]
                stg = stg_v.at[s]

                @plsc.parallel_loop(0, 128, unroll=8)
                def _body(v):
                    vvec = jnp.full((16,), v, jnp.int32)
                    for g in range(4):
                        vec = plsc.load_gather(blk, [rows_g[g], vvec])
                        stg[v, pl.ds(g * 16, 16)] = vec

        def start_put(i, s):
            c = i * NW + wid

            @pl.when(c < TBLK)
            def _():
                pltpu.async_copy(stg_v.at[s],
                                 wp_hbm.at[pl.ds(c * 128, 128)], sems_p[s])

        def wait_put(i, s):
            c = i * NW + wid

            @pl.when(c < TBLK)
            def _():
                pltpu.make_async_copy(stg_v.at[s],
                                      wp_hbm.at[pl.ds(0, 128)],
                                      sems_p[s]).wait()

        def step(i, s, first_pair):
            wait_get(i, s)
            start_get(i + 1, 1 - s)
            if not first_pair:
                wait_put(i - 2, s)
            transpose(i, s)
            start_put(i, s)

        start_get(0, 0)
        step(0, 0, True)
        step(1, 1, True)
        wait_put(0, 0)
        wait_get(2, 0)
        start_get(3, 1)
        transpose(2, 0)
        start_put(2, 0)

        def pair(g, carry):
            i = 2 * g + 1
            step(i, 1, False)
            step(i + 1, 0, False)
            return carry

        lax.fori_loop(1, 121, pair, 0)
        step(243, 1, False)
        step(244, 0, False)
        wait_put(243, 1)
        wait_put(244, 0)

    return k(Wt)


def _sc_gather(idx3, table):
    mesh = plsc.VectorSubcoreMesh(core_axis_name="c", subcore_axis_name="s")

    @functools.partial(
        pl.kernel,
        mesh=mesh,
        compiler_params=pltpu.CompilerParams(use_tc_tiling_on_sc=False),
        out_type=jax.ShapeDtypeStruct((NTOK, 128), jnp.float32),
        scratch_types=[
            pltpu.VMEM((NCHUNK, CB), jnp.int32),
            pltpu.VMEM((NBUF, CB, 128), jnp.float32),
        ] + [pltpu.SemaphoreType.DMA] * (2 * NBUF),
    )
    def k(idx_hbm, w_hbm, out_hbm, idx_v, rows_v, *sems):
        sem_g, sem_p = sems[:NBUF], sems[NBUF:]
        wid = lax.axis_index("s") * NC + lax.axis_index("c")
        base = wid * PER_W
        pltpu.sync_copy(idx_hbm.at[wid], idx_v)

        def gather(j, b):
            pltpu.async_copy(w_hbm.at[idx_v.at[j]], rows_v.at[b], sem_g[b])

        def wait_gather(j, b):
            pltpu.make_async_copy(
                w_hbm.at[idx_v.at[j]], rows_v.at[b], sem_g[b]).wait()

        def put(j, b):
            pltpu.async_copy(
                rows_v.at[b, :, pl.ds(0, DIM)],
                out_hbm.at[pl.ds(base + j * CB, CB), pl.ds(0, DIM)], sem_p[b])

        def wait_put(b):
            pltpu.make_async_copy(
                rows_v.at[b, :, pl.ds(0, DIM)],
                out_hbm.at[pl.ds(base, CB), pl.ds(0, DIM)], sem_p[b]).wait()

        for t in range(LOOK):
            gather(t, t)
        for b in range(NBUF):
            wait_gather(b, b)
            put(b, b)
            jn = b + LOOK
            bn = jn % NBUF
            if jn >= NBUF:
                wait_put(bn)
            gather(jn, bn)

        def group(g, c):
            j0 = g * NBUF
            for b in range(NBUF):
                j = j0 + b
                wait_gather(j, b)
                put(j, b)
                bn = (b + LOOK) % NBUF
                wait_put(bn)
                gather(j + LOOK, bn)
            return c

        lax.fori_loop(1, NGROUP - 1, group, 0)

        j0 = NCHUNK - NBUF
        for b in range(NBUF):
            j = j0 + b
            wait_gather(j, b)
            put(j, b)
            jn = j + LOOK
            if jn < NCHUNK:
                bn = (b + LOOK) % NBUF
                wait_put(bn)
                gather(jn, bn)
        for b in range(NBUF):
            wait_put(b)

    return k(idx3, table)


def kernel(input_word, W):
    idx3 = input_word.reshape(NW, NCHUNK, CB)
    Wp = _sc_detile(W.T)
    out = _sc_gather(idx3, Wp)
    return out[:, :DIM].reshape(B, L, DIM)


# speedup vs baseline: 2.0567x; 1.3023x over previous
import functools

import jax
import jax.numpy as jnp
from jax import lax
from jax.experimental import pallas as pl
from jax.experimental.pallas import tpu as pltpu
from jax.experimental.pallas import tpu_sc as plsc

VOCAB = 1000000
DIM = 64
B, L = 4096, 200
NTOK = B * L

NC, NS = 2, 16
NW = NC * NS
CB = 64
PER_W = NTOK // NW
NCHUNK = PER_W // CB
NBUF = 8
LOOK = 4
NGROUP = NCHUNK // NBUF


def _sc_gather(idx3, table):
    mesh = plsc.VectorSubcoreMesh(core_axis_name="c", subcore_axis_name="s")

    @functools.partial(
        pl.kernel,
        mesh=mesh,
        compiler_params=pltpu.CompilerParams(use_tc_tiling_on_sc=False),
        out_type=jax.ShapeDtypeStruct((NTOK, 128), jnp.float32),
        scratch_types=[
            pltpu.VMEM((NCHUNK, CB), jnp.int32),
            pltpu.VMEM((NBUF, CB, 128), jnp.float32),
        ] + [pltpu.SemaphoreType.DMA] * (2 * NBUF),
    )
    def k(idx_hbm, w_hbm, out_hbm, idx_v, rows_v, *sems):
        sem_g, sem_p = sems[:NBUF], sems[NBUF:]
        wid = lax.axis_index("s") * NC + lax.axis_index("c")
        base = wid * PER_W
        pltpu.sync_copy(idx_hbm.at[wid], idx_v)

        def gather(j, b):
            pltpu.async_copy(w_hbm.at[idx_v.at[j]], rows_v.at[b], sem_g[b])

        def wait_gather(j, b):
            pltpu.make_async_copy(
                w_hbm.at[idx_v.at[j]], rows_v.at[b], sem_g[b]).wait()

        def put(j, b):
            pltpu.async_copy(
                rows_v.at[b, :, pl.ds(0, DIM)],
                out_hbm.at[pl.ds(base + j * CB, CB), pl.ds(0, DIM)], sem_p[b])

        def wait_put(b):
            pltpu.make_async_copy(
                rows_v.at[b, :, pl.ds(0, DIM)],
                out_hbm.at[pl.ds(base, CB), pl.ds(0, DIM)], sem_p[b]).wait()

        for t in range(LOOK):
            gather(t, t)
        for b in range(NBUF):
            wait_gather(b, b)
            put(b, b)
            jn = b + LOOK
            bn = jn % NBUF
            if jn >= NBUF:
                wait_put(bn)
            gather(jn, bn)

        def group(g, c):
            j0 = g * NBUF
            for b in range(NBUF):
                j = j0 + b
                wait_gather(j, b)
                put(j, b)
                bn = (b + LOOK) % NBUF
                wait_put(bn)
                gather(j + LOOK, bn)
            return c

        lax.fori_loop(1, NGROUP - 1, group, 0)

        j0 = NCHUNK - NBUF
        for b in range(NBUF):
            j = j0 + b
            wait_gather(j, b)
            put(j, b)
            jn = j + LOOK
            if jn < NCHUNK:
                bn = (b + LOOK) % NBUF
                wait_put(bn)
                gather(jn, bn)
        for b in range(NBUF):
            wait_put(b)

    return k(idx3, table)


def kernel(input_word, W):
    idx3 = input_word.reshape(NW, NCHUNK, CB)
    Wp = jnp.pad(W, ((0, 0), (0, 128 - DIM)))
    out = _sc_gather(idx3, Wp)
    return out[:, :DIM].reshape(B, L, DIM)


# speedup vs baseline: 2.0665x; 1.0048x over previous
import functools

import jax
import jax.numpy as jnp
from jax import lax
from jax.experimental import pallas as pl
from jax.experimental.pallas import tpu as pltpu
from jax.experimental.pallas import tpu_sc as plsc

VOCAB = 1000000
DIM = 64
B, L = 4096, 200
NTOK = B * L

NC, NS = 2, 16
NW = NC * NS
CB = 64
PER_W = NTOK // NW
NCHUNK = PER_W // CB
NBUF = 8
LOOK = 6
NGROUP = NCHUNK // NBUF


def _sc_gather(idx3, table):
    mesh = plsc.VectorSubcoreMesh(core_axis_name="c", subcore_axis_name="s")

    @functools.partial(
        pl.kernel,
        mesh=mesh,
        compiler_params=pltpu.CompilerParams(use_tc_tiling_on_sc=False),
        out_type=jax.ShapeDtypeStruct((NTOK, 128), jnp.float32),
        scratch_types=[
            pltpu.VMEM((NCHUNK, CB), jnp.int32),
            pltpu.VMEM((NBUF, CB, 128), jnp.float32),
        ] + [pltpu.SemaphoreType.DMA] * (2 * NBUF),
    )
    def k(idx_hbm, w_hbm, out_hbm, idx_v, rows_v, *sems):
        sem_g, sem_p = sems[:NBUF], sems[NBUF:]
        wid = lax.axis_index("s") * NC + lax.axis_index("c")
        base = wid * PER_W
        pltpu.sync_copy(idx_hbm.at[wid], idx_v)

        def gather(j, b):
            pltpu.async_copy(w_hbm.at[idx_v.at[j]], rows_v.at[b], sem_g[b])

        def wait_gather(j, b):
            pltpu.make_async_copy(
                w_hbm.at[idx_v.at[j]], rows_v.at[b], sem_g[b]).wait()

        def put(j, b):
            pltpu.async_copy(
                rows_v.at[b, :, pl.ds(0, DIM)],
                out_hbm.at[pl.ds(base + j * CB, CB), pl.ds(0, DIM)], sem_p[b])

        def wait_put(b):
            pltpu.make_async_copy(
                rows_v.at[b, :, pl.ds(0, DIM)],
                out_hbm.at[pl.ds(base, CB), pl.ds(0, DIM)], sem_p[b]).wait()

        for t in range(LOOK):
            gather(t, t)
        for b in range(NBUF):
            wait_gather(b, b)
            put(b, b)
            jn = b + LOOK
            bn = jn % NBUF
            if jn >= NBUF:
                wait_put(bn)
            gather(jn, bn)

        def group(g, c):
            j0 = g * NBUF
            for b in range(NBUF):
                j = j0 + b
                wait_gather(j, b)
                put(j, b)
                bn = (b + LOOK) % NBUF
                wait_put(bn)
                gather(j + LOOK, bn)
            return c

        lax.fori_loop(1, NGROUP - 1, group, 0)

        j0 = NCHUNK - NBUF
        for b in range(NBUF):
            j = j0 + b
            wait_gather(j, b)
            put(j, b)
            jn = j + LOOK
            if jn < NCHUNK:
                bn = (b + LOOK) % NBUF
                wait_put(bn)
                gather(jn, bn)
        for b in range(NBUF):
            wait_put(b)

    return k(idx3, table)


def kernel(input_word, W):
    idx3 = input_word.reshape(NW, NCHUNK, CB)
    Wp = jnp.pad(W, ((0, 0), (0, 128 - DIM)))
    out = _sc_gather(idx3, Wp)
    return out[:, :DIM].reshape(B, L, DIM)
